# baseline (device time: 8287 ns/iter reference)
import jax
import jax.numpy as jnp
from jax import lax
from jax.experimental import pallas as pl
from jax.experimental.pallas import tpu as pltpu

N_CHUNKS = 4


def kernel(x, dy, gamma):
    m_per, d = x.shape
    rows = m_per // N_CHUNKS

    def body(x_hbm, dy_hbm, gamma_hbm, out_ref,
             xbuf, dybuf, xsems, dysems, comm_ref, send_sem, recv_sem):
        my_x = lax.axis_index("x")
        my_y = lax.axis_index("y")
        my_z = lax.axis_index("z")
        peer = (my_x, 1 - my_y, my_z)

        barrier_sem = pltpu.get_barrier_semaphore()
        pl.semaphore_signal(
            barrier_sem, inc=1, device_id=peer,
            device_id_type=pl.DeviceIdType.MESH,
        )

        def chunk_copies(k):
            slot = k % 2
            cx = pltpu.make_async_copy(
                x_hbm.at[pl.ds(k * rows, rows), :], xbuf.at[slot],
                xsems.at[slot])
            cd = pltpu.make_async_copy(
                dy_hbm.at[pl.ds(k * rows, rows), :], dybuf.at[slot],
                dysems.at[slot])
            return cx, cd

        pending = {0: chunk_copies(0)}
        for c in pending[0]:
            c.start()

        dgamma = jnp.zeros((1, d), jnp.float32)
        dbeta = jnp.zeros((1, d), jnp.float32)
        for k in range(N_CHUNKS):
            if k + 1 < N_CHUNKS:
                pending[k + 1] = chunk_copies(k + 1)
                for c in pending[k + 1]:
                    c.start()
            for c in pending.pop(k):
                c.wait()
            xv = xbuf[k % 2]
            dyv = dybuf[k % 2]
            mu = jnp.mean(xv, axis=1, keepdims=True)
            var = jnp.mean((xv - mu) * (xv - mu), axis=1, keepdims=True)
            rstd = lax.rsqrt(var + 1e-5)
            xhat = (xv - mu) * rstd
            dgamma = dgamma + jnp.sum(dyv * xhat, axis=0, keepdims=True)
            dbeta = dbeta + jnp.sum(dyv, axis=0, keepdims=True)

        comm_ref[0, :, :] = jnp.concatenate([dgamma, dbeta], axis=0)

        pl.semaphore_wait(barrier_sem, 1)

        rdma = pltpu.make_async_remote_copy(
            src_ref=comm_ref.at[0],
            dst_ref=comm_ref.at[1],
            send_sem=send_sem,
            recv_sem=recv_sem,
            device_id=peer,
            device_id_type=pl.DeviceIdType.MESH,
        )
        rdma.start()
        rdma.wait()

        out_ref[:, :] = comm_ref[0, :, :] + comm_ref[1, :, :]

    return pl.pallas_call(
        body,
        out_shape=jax.ShapeDtypeStruct((2, d), jnp.float32),
        in_specs=[
            pl.BlockSpec(memory_space=pl.ANY),
            pl.BlockSpec(memory_space=pl.ANY),
            pl.BlockSpec(memory_space=pl.ANY),
        ],
        out_specs=pl.BlockSpec(memory_space=pltpu.VMEM),
        scratch_shapes=[
            pltpu.VMEM((2, rows, d), jnp.float32),
            pltpu.VMEM((2, rows, d), jnp.float32),
            pltpu.SemaphoreType.DMA((2,)),
            pltpu.SemaphoreType.DMA((2,)),
            pltpu.VMEM((2, 2, d), jnp.float32),
            pltpu.SemaphoreType.DMA,
            pltpu.SemaphoreType.DMA,
        ],
        compiler_params=pltpu.CompilerParams(collective_id=0),
    )(x, dy, gamma)


# device time: 6883 ns/iter; 1.2040x vs baseline; 1.2040x over previous
import jax
import jax.numpy as jnp
from jax import lax
from jax.experimental import pallas as pl
from jax.experimental.pallas import tpu as pltpu

N_CHUNKS = 4


def kernel(x, dy, gamma):
    m_per, d = x.shape
    rows = m_per // N_CHUNKS

    def body(x_hbm, dy_hbm, gamma_hbm, out_ref,
             xbuf, dybuf, xsems, dysems, comm_ref, send_sem, recv_sem):
        my_x = lax.axis_index("x")
        my_y = lax.axis_index("y")
        my_z = lax.axis_index("z")
        peer = (my_x, 1 - my_y, my_z)

        barrier_sem = pltpu.get_barrier_semaphore()
        pl.semaphore_signal(
            barrier_sem, inc=1, device_id=peer,
            device_id_type=pl.DeviceIdType.MESH,
        )

        def chunk_copies(k):
            slot = k % 2
            cx = pltpu.make_async_copy(
                x_hbm.at[pl.ds(k * rows, rows), :], xbuf.at[slot],
                xsems.at[slot])
            cd = pltpu.make_async_copy(
                dy_hbm.at[pl.ds(k * rows, rows), :], dybuf.at[slot],
                dysems.at[slot])
            return cx, cd

        pending = {0: chunk_copies(0)}
        for c in pending[0]:
            c.start()

        dgamma = jnp.zeros((1, d), jnp.float32)
        dbeta = jnp.zeros((1, d), jnp.float32)
        for k in range(N_CHUNKS):
            if k + 1 < N_CHUNKS:
                pending[k + 1] = chunk_copies(k + 1)
                for c in pending[k + 1]:
                    c.start()
            for c in pending.pop(k):
                c.wait()
            xv = xbuf[k % 2]
            dyv = dybuf[k % 2]
            mu = jnp.mean(xv, axis=1, keepdims=True)
            var = jnp.mean((xv - mu) * (xv - mu), axis=1, keepdims=True)
            rstd = lax.rsqrt(var + 1e-5)
            xhat = (xv - mu) * rstd
            dgamma = dgamma + jnp.sum(dyv * xhat, axis=0, keepdims=True)
            dbeta = dbeta + jnp.sum(dyv, axis=0, keepdims=True)

        comm_ref[0, :, :] = jnp.concatenate([dgamma, dbeta], axis=0)

        pl.semaphore_wait(barrier_sem, 1)

        rdma = pltpu.make_async_remote_copy(
            src_ref=comm_ref.at[0],
            dst_ref=comm_ref.at[1],
            send_sem=send_sem,
            recv_sem=recv_sem,
            device_id=peer,
            device_id_type=pl.DeviceIdType.MESH,
        )
        rdma.start()
        rdma.wait()

        out_ref[:, :] = comm_ref[0, :, :] + comm_ref[1, :, :]

    return pl.pallas_call(
        body,
        out_shape=jax.ShapeDtypeStruct((2, d), jnp.float32),
        in_specs=[
            pl.BlockSpec(memory_space=pl.ANY),
            pl.BlockSpec(memory_space=pl.ANY),
            pl.BlockSpec(memory_space=pl.ANY),
        ],
        out_specs=pl.BlockSpec(memory_space=pltpu.VMEM),
        scratch_shapes=[
            pltpu.VMEM((2, rows, d), jnp.float32),
            pltpu.VMEM((2, rows, d), jnp.float32),
            pltpu.SemaphoreType.DMA((2,)),
            pltpu.SemaphoreType.DMA((2,)),
            pltpu.VMEM((2, 2, d), jnp.float32),
            pltpu.SemaphoreType.DMA,
            pltpu.SemaphoreType.DMA,
        ],
        compiler_params=pltpu.CompilerParams(collective_id=0),
    )(
        pltpu.with_memory_space_constraint(x, pltpu.MemorySpace.HBM),
        pltpu.with_memory_space_constraint(dy, pltpu.MemorySpace.HBM),
        pltpu.with_memory_space_constraint(gamma, pltpu.MemorySpace.HBM),
    )


# device time: 5909 ns/iter; 1.4024x vs baseline; 1.1648x over previous
import jax
import jax.numpy as jnp
from jax import lax
from jax.experimental import pallas as pl
from jax.experimental.pallas import tpu as pltpu

N_CHUNKS = 2


def kernel(x, dy, gamma):
    m_per, d = x.shape
    rows = m_per // N_CHUNKS

    def body(x_hbm, dy_hbm, gamma_hbm, out_ref,
             xbuf, dybuf, xsems, dysems, comm_ref, send_sem, recv_sem,
             res_ref, out_sem):
        my_x = lax.axis_index("x")
        my_y = lax.axis_index("y")
        my_z = lax.axis_index("z")
        peer = (my_x, 1 - my_y, my_z)

        barrier_sem = pltpu.get_barrier_semaphore()
        pl.semaphore_signal(
            barrier_sem, inc=1, device_id=peer,
            device_id_type=pl.DeviceIdType.MESH,
        )

        def chunk_copies(k):
            slot = k % 2
            cx = pltpu.make_async_copy(
                x_hbm.at[pl.ds(k * rows, rows), :], xbuf.at[slot],
                xsems.at[slot])
            cd = pltpu.make_async_copy(
                dy_hbm.at[pl.ds(k * rows, rows), :], dybuf.at[slot],
                dysems.at[slot])
            return cx, cd

        pending = {0: chunk_copies(0)}
        for c in pending[0]:
            c.start()

        dgamma = jnp.zeros((1, d), jnp.float32)
        dbeta = jnp.zeros((1, d), jnp.float32)
        for k in range(N_CHUNKS):
            if k + 1 < N_CHUNKS:
                pending[k + 1] = chunk_copies(k + 1)
                for c in pending[k + 1]:
                    c.start()
            for c in pending.pop(k):
                c.wait()
            xv = xbuf[k % 2]
            dyv = dybuf[k % 2]
            mu = jnp.mean(xv, axis=1, keepdims=True)
            var = jnp.mean((xv - mu) * (xv - mu), axis=1, keepdims=True)
            rstd = lax.rsqrt(var + 1e-5)
            xhat = (xv - mu) * rstd
            dgamma = dgamma + jnp.sum(dyv * xhat, axis=0, keepdims=True)
            dbeta = dbeta + jnp.sum(dyv, axis=0, keepdims=True)

        comm_ref[0, :, :] = jnp.concatenate([dgamma, dbeta], axis=0)

        pl.semaphore_wait(barrier_sem, 1)

        rdma = pltpu.make_async_remote_copy(
            src_ref=comm_ref.at[0],
            dst_ref=comm_ref.at[1],
            send_sem=send_sem,
            recv_sem=recv_sem,
            device_id=peer,
            device_id_type=pl.DeviceIdType.MESH,
        )
        rdma.start()
        rdma.wait()

        res_ref[:, :] = comm_ref[0, :, :] + comm_ref[1, :, :]
        out_copy = pltpu.make_async_copy(res_ref, out_ref, out_sem)
        out_copy.start()
        out_copy.wait()

    return pl.pallas_call(
        body,
        out_shape=jax.ShapeDtypeStruct((2, d), jnp.float32),
        in_specs=[
            pl.BlockSpec(memory_space=pl.ANY),
            pl.BlockSpec(memory_space=pl.ANY),
            pl.BlockSpec(memory_space=pl.ANY),
        ],
        out_specs=pl.BlockSpec(memory_space=pl.ANY),
        scratch_shapes=[
            pltpu.VMEM((2, rows, d), jnp.float32),
            pltpu.VMEM((2, rows, d), jnp.float32),
            pltpu.SemaphoreType.DMA((2,)),
            pltpu.SemaphoreType.DMA((2,)),
            pltpu.VMEM((2, 2, d), jnp.float32),
            pltpu.SemaphoreType.DMA,
            pltpu.SemaphoreType.DMA,
            pltpu.VMEM((2, d), jnp.float32),
            pltpu.SemaphoreType.DMA,
        ],
        compiler_params=pltpu.CompilerParams(collective_id=0),
    )(
        pltpu.with_memory_space_constraint(x, pltpu.MemorySpace.HBM),
        pltpu.with_memory_space_constraint(dy, pltpu.MemorySpace.HBM),
        pltpu.with_memory_space_constraint(gamma, pltpu.MemorySpace.HBM),
    )


# device time: 5897 ns/iter; 1.4053x vs baseline; 1.0020x over previous
import jax
import jax.numpy as jnp
from jax import lax
from jax.experimental import pallas as pl
from jax.experimental.pallas import tpu as pltpu

N_CHUNKS = 2


def kernel(x, dy, gamma):
    m_per, d = x.shape
    rows = m_per // N_CHUNKS

    def body(x_hbm, dy_hbm, gamma_hbm, out_ref,
             xbuf, dybuf, xsems, dysems, comm_ref, send_sem, recv_sem):
        my_x = lax.axis_index("x")
        my_y = lax.axis_index("y")
        my_z = lax.axis_index("z")
        peer = (my_x, 1 - my_y, my_z)

        barrier_sem = pltpu.get_barrier_semaphore()
        pl.semaphore_signal(
            barrier_sem, inc=1, device_id=peer,
            device_id_type=pl.DeviceIdType.MESH,
        )

        def chunk_copies(k):
            slot = k % 2
            cx = pltpu.make_async_copy(
                x_hbm.at[pl.ds(k * rows, rows), :], xbuf.at[slot],
                xsems.at[slot])
            cd = pltpu.make_async_copy(
                dy_hbm.at[pl.ds(k * rows, rows), :], dybuf.at[slot],
                dysems.at[slot])
            return cx, cd

        pending = {0: chunk_copies(0)}
        for c in pending[0]:
            c.start()

        dgamma = jnp.zeros((1, d), jnp.float32)
        dbeta = jnp.zeros((1, d), jnp.float32)
        for k in range(N_CHUNKS):
            if k + 1 < N_CHUNKS:
                pending[k + 1] = chunk_copies(k + 1)
                for c in pending[k + 1]:
                    c.start()
            for c in pending.pop(k):
                c.wait()
            xv = xbuf[k % 2]
            dyv = dybuf[k % 2]
            mu = jnp.mean(xv, axis=1, keepdims=True)
            var = jnp.mean((xv - mu) * (xv - mu), axis=1, keepdims=True)
            rstd = lax.rsqrt(var + 1e-5)
            xhat = (xv - mu) * rstd
            dgamma = dgamma + jnp.sum(dyv * xhat, axis=0, keepdims=True)
            dbeta = dbeta + jnp.sum(dyv, axis=0, keepdims=True)

        comm_ref[0, :, :] = jnp.concatenate([dgamma, dbeta], axis=0)

        pl.semaphore_wait(barrier_sem, 1)

        rdma = pltpu.make_async_remote_copy(
            src_ref=comm_ref.at[0],
            dst_ref=comm_ref.at[1],
            send_sem=send_sem,
            recv_sem=recv_sem,
            device_id=peer,
            device_id_type=pl.DeviceIdType.MESH,
        )
        rdma.start()
        rdma.wait()

        out_ref[:, :] = comm_ref[0, :, :] + comm_ref[1, :, :]

    return pl.pallas_call(
        body,
        out_shape=jax.ShapeDtypeStruct((2, d), jnp.float32),
        in_specs=[
            pl.BlockSpec(memory_space=pl.ANY),
            pl.BlockSpec(memory_space=pl.ANY),
            pl.BlockSpec(memory_space=pl.ANY),
        ],
        out_specs=pl.BlockSpec(memory_space=pltpu.VMEM),
        scratch_shapes=[
            pltpu.VMEM((2, rows, d), jnp.float32),
            pltpu.VMEM((2, rows, d), jnp.float32),
            pltpu.SemaphoreType.DMA((2,)),
            pltpu.SemaphoreType.DMA((2,)),
            pltpu.VMEM((2, 2, d), jnp.float32),
            pltpu.SemaphoreType.DMA,
            pltpu.SemaphoreType.DMA,
        ],
        compiler_params=pltpu.CompilerParams(collective_id=0),
    )(
        pltpu.with_memory_space_constraint(x, pltpu.MemorySpace.HBM),
        pltpu.with_memory_space_constraint(dy, pltpu.MemorySpace.HBM),
        pltpu.with_memory_space_constraint(gamma, pltpu.MemorySpace.HBM),
    )
